# Initial kernel scaffold; baseline (speedup 1.0000x reference)
import jax
import jax.numpy as jnp
from jax.experimental import pallas as pl


def kernel(x, W_enc, b_enc, c_buffer, z_buffer, r0_buffer, tau_buffer, W1, b1, W2, b2, t_buffer):
    raise NotImplementedError("write your pallas kernel here")


# speedup vs baseline: 1.0054x; 1.0054x over previous
import functools

import jax
import jax.numpy as jnp
from jax import lax
from jax.experimental import pallas as pl
from jax.experimental.pallas import tpu as pltpu

_B = 1024
_DIN = 512
_D = 256
_CAP = 65536
_K = 32
_CHUNK = 2048
_NCH = _CAP // _CHUNK
_GRP = 128
_NGRP = _CHUNK // _GRP
_DECAY = 0.01


def _scores_body(x_ref, we_ref, be_ref, c_ref, r0_ref, tau_ref, t_ref,
                 s_out, m_out, z_out, zn_scr):
    j = pl.program_id(0)

    @pl.when(j == 0)
    def _():
        z = jnp.dot(x_ref[...], we_ref[...],
                    preferred_element_type=jnp.float32)
        z = jnp.maximum(z + be_ref[...], 0.0)
        z_out[...] = z
        n = jnp.sqrt(jnp.sum(z * z, axis=1, keepdims=True))
        zn_scr[...] = z / jnp.maximum(n, 1e-8)

    c = c_ref[...]
    n2 = jnp.sum(c * c, axis=1, keepdims=True)
    cn = c / jnp.maximum(jnp.sqrt(n2), 1e-8)
    s = lax.dot_general(zn_scr[...], cn, (((1,), (1,)), ((), ())),
                        preferred_element_type=jnp.float32)
    sal = r0_ref[...] * jnp.exp(-_DECAY * tau_ref[...])
    sal = sal * (t_ref[...] != 0).astype(jnp.float32)
    s = s * sal
    s_out[...] = s
    m_out[0] = jnp.max(s.reshape(_B, _NGRP, _GRP), axis=2)


def _scores_call(x, W_enc, b_enc, c_buffer, r0, tau, t):
    return pl.pallas_call(
        _scores_body,
        grid=(_NCH,),
        in_specs=[
            pl.BlockSpec((_B, _DIN), lambda j: (0, 0)),
            pl.BlockSpec((_DIN, _D), lambda j: (0, 0)),
            pl.BlockSpec((1, _D), lambda j: (0, 0)),
            pl.BlockSpec((_CHUNK, _D), lambda j: (j, 0)),
            pl.BlockSpec((1, _CHUNK), lambda j: (0, j)),
            pl.BlockSpec((1, _CHUNK), lambda j: (0, j)),
            pl.BlockSpec((1, _CHUNK), lambda j: (0, j)),
        ],
        out_specs=[
            pl.BlockSpec((_B, _CHUNK), lambda j: (0, j)),
            pl.BlockSpec((1, _B, _NGRP), lambda j: (j, 0, 0)),
            pl.BlockSpec((_B, _D), lambda j: (0, 0)),
        ],
        out_shape=[
            jax.ShapeDtypeStruct((_B, _CAP), jnp.float32),
            jax.ShapeDtypeStruct((_NCH, _B, _NGRP), jnp.float32),
            jax.ShapeDtypeStruct((_B, _D), jnp.float32),
        ],
        scratch_shapes=[pltpu.VMEM((_B, _D), jnp.float32)],
    )(x, W_enc, b_enc.reshape(1, _D), c_buffer,
      r0.reshape(1, _CAP), tau.reshape(1, _CAP), t.reshape(1, _CAP))


def kernel(x, W_enc, b_enc, c_buffer, z_buffer, r0_buffer, tau_buffer,
           W1, b1, W2, b2, t_buffer):
    scores, _m3, z = _scores_call(x, W_enc, b_enc, c_buffer,
                                  r0_buffer, tau_buffer, t_buffer)
    top_vals, top_idx = lax.top_k(scores, _K)
    z_topk = jnp.take(z_buffer, top_idx, axis=0)
    w = top_vals / (jnp.sum(top_vals, axis=-1, keepdims=True) + 1e-8)
    r = jnp.sum(w[..., None] * z_topk, axis=1)
    h = jax.nn.relu(jnp.concatenate([z, r], axis=-1) @ W1 + b1)
    return h @ W2 + b2


# speedup vs baseline: 8.4845x; 8.4392x over previous
import functools

import jax
import jax.numpy as jnp
from jax import lax
from jax.experimental import pallas as pl
from jax.experimental.pallas import tpu as pltpu
from jax.experimental.pallas import tpu_sc as plsc

_B = 1024
_DIN = 512
_D = 256
_CAP = 65536
_K = 32
_CHUNK = 2048
_NCH = _CAP // _CHUNK
_GRP = 128
_NGRP = _CHUNK // _GRP
_DECAY = 0.01


def _scores_body(x_ref, we_ref, be_ref, c_ref, r0_ref, tau_ref, t_ref,
                 s_out, m_out, z_out, zn_scr):
    j = pl.program_id(0)

    @pl.when(j == 0)
    def _():
        z = jnp.dot(x_ref[...], we_ref[...],
                    preferred_element_type=jnp.float32)
        z = jnp.maximum(z + be_ref[...], 0.0)
        z_out[...] = z
        n = jnp.sqrt(jnp.sum(z * z, axis=1, keepdims=True))
        zn_scr[...] = z / jnp.maximum(n, 1e-8)

    c = c_ref[...]
    n2 = jnp.sum(c * c, axis=1, keepdims=True)
    cn = c / jnp.maximum(jnp.sqrt(n2), 1e-8)
    s = lax.dot_general(zn_scr[...], cn, (((1,), (1,)), ((), ())),
                        preferred_element_type=jnp.float32)
    sal = r0_ref[...] * jnp.exp(-_DECAY * tau_ref[...])
    sal = sal * (t_ref[...] != 0).astype(jnp.float32)
    s = s * sal
    s_out[...] = s
    m_out[0] = jnp.max(s.reshape(_B, _NGRP, _GRP), axis=2)


def _scores_call(x, W_enc, b_enc, c_buffer, r0, tau, t):
    return pl.pallas_call(
        _scores_body,
        grid=(_NCH,),
        in_specs=[
            pl.BlockSpec((_B, _DIN), lambda j: (0, 0)),
            pl.BlockSpec((_DIN, _D), lambda j: (0, 0)),
            pl.BlockSpec((1, _D), lambda j: (0, 0)),
            pl.BlockSpec((_CHUNK, _D), lambda j: (j, 0)),
            pl.BlockSpec((1, _CHUNK), lambda j: (0, j)),
            pl.BlockSpec((1, _CHUNK), lambda j: (0, j)),
            pl.BlockSpec((1, _CHUNK), lambda j: (0, j)),
        ],
        out_specs=[
            pl.BlockSpec((_B, _CHUNK), lambda j: (0, j)),
            pl.BlockSpec((1, _B, _NGRP), lambda j: (j, 0, 0)),
            pl.BlockSpec((_B, _D), lambda j: (0, 0)),
        ],
        out_shape=[
            jax.ShapeDtypeStruct((_B, _CAP), jnp.float32),
            jax.ShapeDtypeStruct((_NCH, _B, _NGRP), jnp.float32),
            jax.ShapeDtypeStruct((_B, _D), jnp.float32),
        ],
        scratch_shapes=[pltpu.VMEM((_B, _D), jnp.float32)],
    )(x, W_enc, b_enc.reshape(1, _D), c_buffer,
      r0.reshape(1, _CAP), tau.reshape(1, _CAP), t.reshape(1, _CAP))


_NEG = float(-3.0e38)
_NCHK = _CAP // _GRP


def _thresh_body(m_ref, t_ref):
    v = m_ref[...]
    for _ in range(_K - 1):
        mx = jnp.max(v, axis=1, keepdims=True)
        v = jnp.where(v == mx, _NEG, v)
    t_ref[...] = jnp.max(v, axis=1, keepdims=True)


def _thresh_call(m2d):
    return pl.pallas_call(
        _thresh_body,
        out_shape=jax.ShapeDtypeStruct((_B, 1), jnp.float32),
    )(m2d)


_NC = 2
_NS = 16
_NW = _NC * _NS
_RPW = _B // _NW
_SELCAP = 48
_CCAP = 256


def _sc_body(s2_hbm, m_hbm, t_hbm, zb_hbm, r_hbm,
             m_v, t_v, pad_v, sel_v, gbuf_v, cval_v, cloc_v,
             wval_v, wloc_v, zidx_v, zrows_v, racc_v, sem1, sem2):
    wid = lax.axis_index("s") * _NC + lax.axis_index("c")
    wbase = wid * _RPW
    iota16 = lax.iota(jnp.int32, 16)
    pltpu.sync_copy(m_hbm.at[pl.ds(wbase * _NCHK, _RPW * _NCHK)], m_v)
    pltpu.sync_copy(t_hbm.at[pl.ds(wbase, _RPW)], t_v)

    def row_body(i, _unused):
        b = wbase + i
        grow = b * _NCHK
        tvec = t_v[pl.ds((i >> 4) * 16, 16)]
        tb = jnp.max(jnp.where(iota16 == (i & 15), tvec, _NEG))

        for q in range(_SELCAP // 16 + 1):
            pad_v[pl.ds(q * 16, 16)] = jnp.full((16,), grow, jnp.int32)

        def selchunk(j, cnt):
            m = m_v[pl.ds(i * _NCHK + j * 16, 16)]
            msk = m >= tb
            cidx = iota16 + (grow + j * 16)
            plsc.store_compressed(pad_v.at[pl.ds(cnt, 16)], cidx, mask=msk)
            return jnp.minimum(cnt + jnp.sum(msk.astype(jnp.int32)), _SELCAP)

        nsel = lax.fori_loop(0, _NCHK // 16, selchunk, 0)
        for q in range(_SELCAP // 16):
            sel_v[pl.ds(q * 16, 16)] = pad_v[pl.ds(q * 16, 16)]

        pltpu.async_copy(s2_hbm.at[sel_v], gbuf_v, sem1).wait()

        def rescan(k, cnt):
            for v in range(_GRP // 16):
                s = gbuf_v[k, pl.ds(v * 16, 16)]
                msk = s >= tb
                loc = iota16 + (k * _GRP + v * 16)
                plsc.store_compressed(cval_v.at[pl.ds(cnt, 16)], s, mask=msk)
                plsc.store_compressed(cloc_v.at[pl.ds(cnt, 16)], loc, mask=msk)
                cnt = jnp.minimum(cnt + jnp.sum(msk.astype(jnp.int32)), _CCAP)
            return cnt

        ncand = lax.fori_loop(0, nsel, rescan, 0)
        cval_v[pl.ds(ncand, 16)] = jnp.full((16,), _NEG, jnp.float32)
        cloc_v[pl.ds(ncand, 16)] = jnp.zeros((16,), jnp.int32)

        nv = (ncand + 15) >> 4

        def select_k(kk, _):
            def scanv(vi, carry):
                bv, bloc, bpos = carry
                mv_ = cval_v[pl.ds(vi * 16, 16)]
                lv_ = cloc_v[pl.ds(vi * 16, 16)]
                pv_ = iota16 + vi * 16
                upd = mv_ > bv
                return (jnp.where(upd, mv_, bv), jnp.where(upd, lv_, bloc),
                        jnp.where(upd, pv_, bpos))

            z16i = jnp.zeros((16,), jnp.int32)
            bv, bloc, bpos = lax.fori_loop(
                0, nv, scanv,
                (jnp.full((16,), _NEG, jnp.float32), z16i, z16i))
            mx = jnp.max(bv)
            lm = bv == mx
            big = jnp.int32(2 ** 30)
            wloc = jnp.min(jnp.where(lm, bloc, big))
            wpos = jnp.min(jnp.where(lm, bpos, big))
            m0 = iota16 == 0
            kkv = jnp.full((16,), kk, jnp.int32)
            plsc.store_scatter(wval_v, [kkv],
                               jnp.full((16,), mx, jnp.float32), mask=m0)
            plsc.store_scatter(wloc_v, [kkv],
                               jnp.full((16,), wloc, jnp.int32), mask=m0)
            plsc.store_scatter(cval_v, [jnp.full((16,), wpos, jnp.int32)],
                               jnp.full((16,), _NEG, jnp.float32), mask=m0)
            return 0

        lax.fori_loop(0, _K, select_k, 0)

        wv0 = wval_v[pl.ds(0, 16)]
        wv1 = wval_v[pl.ds(16, 16)]
        den = jnp.sum(wv0) + jnp.sum(wv1) + jnp.float32(1e-8)
        wt0 = wv0 / den
        wt1 = wv1 / den
        l0 = wloc_v[pl.ds(0, 16)]
        l1 = wloc_v[pl.ds(16, 16)]
        c0 = (plsc.load_gather(sel_v, [l0 >> 7]) - grow) * _GRP + (l0 & 127)
        c1 = (plsc.load_gather(sel_v, [l1 >> 7]) - grow) * _GRP + (l1 & 127)
        zidx_v[pl.ds(0, 16)] = c0
        zidx_v[pl.ds(16, 16)] = c1

        pltpu.async_copy(zb_hbm.at[zidx_v], zrows_v, sem2).wait()

        def recall(k, acc):
            cond = jnp.broadcast_to(k < 16, (16,))
            wvec = jnp.where(cond, wt0, wt1)
            wk = jnp.max(jnp.where(iota16 == (k & 15), wvec, _NEG))
            return tuple(acc[t] + wk * zrows_v[k, pl.ds(t * 16, 16)]
                         for t in range(_D // 16))

        zero16 = jnp.zeros((16,), jnp.float32)
        acc = lax.fori_loop(0, _K, recall, (zero16,) * (_D // 16))
        for t in range(_D // 16):
            racc_v[pl.ds(t * 16, 16)] = acc[t]
        pltpu.sync_copy(racc_v, r_hbm.at[b])
        return 0

    lax.fori_loop(0, _RPW, row_body, 0)


@functools.partial(
    pl.kernel,
    out_type=jax.ShapeDtypeStruct((_B, _D), jnp.float32),
    mesh=plsc.VectorSubcoreMesh(core_axis_name="c", subcore_axis_name="s"),
    compiler_params=pltpu.CompilerParams(needs_layout_passes=False),
    scratch_types=[
        pltpu.VMEM((_RPW * _NCHK,), jnp.float32),
        pltpu.VMEM((_RPW,), jnp.float32),
        pltpu.VMEM((_SELCAP + 16,), jnp.int32),
        pltpu.VMEM((_SELCAP,), jnp.int32),
        pltpu.VMEM((_SELCAP, _GRP), jnp.float32),
        pltpu.VMEM((_CCAP + 16,), jnp.float32),
        pltpu.VMEM((_CCAP + 16,), jnp.int32),
        pltpu.VMEM((_K,), jnp.float32),
        pltpu.VMEM((_K,), jnp.int32),
        pltpu.VMEM((_K,), jnp.int32),
        pltpu.VMEM((_K, _D), jnp.float32),
        pltpu.VMEM((_D,), jnp.float32),
        pltpu.SemaphoreType.DMA,
        pltpu.SemaphoreType.DMA,
    ],
)
def _sc_recall(s2_hbm, m_hbm, t_hbm, zb_hbm, r_hbm, *rest):
    _sc_body(s2_hbm, m_hbm, t_hbm, zb_hbm, r_hbm, *rest)


def _dec_body(z_ref, r_ref, w1a_ref, w1b_ref, b1_ref, w2_ref, b2_ref, o_ref):
    h = jnp.dot(z_ref[...], w1a_ref[...], preferred_element_type=jnp.float32)
    h = h + jnp.dot(r_ref[...], w1b_ref[...], preferred_element_type=jnp.float32)
    h = jnp.maximum(h + b1_ref[...], 0.0)
    o_ref[...] = jnp.dot(h, w2_ref[...],
                         preferred_element_type=jnp.float32) + b2_ref[...]


def _dec_call(z, r, W1, b1, W2, b2):
    return pl.pallas_call(
        _dec_body,
        out_shape=jax.ShapeDtypeStruct((_B, _CLS), jnp.float32),
    )(z, r, W1[:_D], W1[_D:], b1.reshape(1, -1), W2, b2.reshape(1, -1))


_CLS = 1000


def kernel(x, W_enc, b_enc, c_buffer, z_buffer, r0_buffer, tau_buffer,
           W1, b1, W2, b2, t_buffer):
    scores, m3, z = _scores_call(x, W_enc, b_enc, c_buffer,
                                 r0_buffer, tau_buffer, t_buffer)
    m2d = m3.transpose(1, 0, 2).reshape(_B, _NCH * _NGRP)
    t = _thresh_call(m2d)
    r = _sc_recall(scores.reshape(_B * _NCHK, _GRP), m2d.reshape(-1),
                   t.reshape(-1), z_buffer)
    h = _dec_call(z, r, W1, b1, W2, b2)
    return h
